# baseline (speedup 1.0000x reference)
import jax
import jax.numpy as jnp
from jax.experimental import pallas as pl
from jax.experimental.pallas import tpu as pltpu

EMBD = 192
NEXP = 8
NSEL = 2
HIDD = 2 * EMBD
BATCH = 32
HGT = 14
WID = 14
HW = HGT * WID
EPS = 1e-5
SPB = 8


def _gate_kernel(x_ref, gw_ref, gb_ref, idx_ref, wts_ref, aux_ref):
    xb = x_ref[...]
    gap = jnp.mean(xb, axis=2)
    logits = jax.lax.dot_general(
        gap, gw_ref[...], (((1,), (1,)), ((), ())),
        preferred_element_type=jnp.float32)
    logits = logits + gb_ref[...]
    m = jnp.max(logits, axis=1, keepdims=True)
    p = jnp.exp(logits - m)
    p = p / jnp.sum(p, axis=1, keepdims=True)
    iota = jax.lax.broadcasted_iota(jnp.int32, (BATCH, NEXP), 1)
    v1 = jnp.max(p, axis=1, keepdims=True)
    i1 = jnp.min(jnp.where(p == v1, iota, NEXP), axis=1, keepdims=True)
    p2 = jnp.where(iota == i1, -1.0, p)
    v2 = jnp.max(p2, axis=1, keepdims=True)
    i2 = jnp.min(jnp.where(p2 == v2, iota, NEXP), axis=1, keepdims=True)
    denom = v1 + v2 + 1e-8
    wa = v1 / denom
    wb = v2 / denom
    idx_ref[:, 0:1] = i1
    idx_ref[:, 1:2] = i2
    wts_ref[:, 0:1] = wa
    wts_ref[:, 1:2] = wb
    gated = jnp.where(iota == i1, wa, 0.0) + jnp.where(iota == i2, wb, 0.0)
    usage = jnp.sum(gated, axis=0, keepdims=True)
    mu = jnp.mean(usage, axis=1, keepdims=True)
    var = jnp.mean((usage - mu) ** 2, axis=1, keepdims=True)
    aux_ref[...] = var / (mu * mu + 1e-10)


def _experts_kernel(idx_ref, gwt_ref, x_ref,
                    w1_ref, b1_ref, g1_ref, be1_ref, rm1_ref, rv1_ref,
                    w2_ref, b2_ref, g2_ref, be2_ref, rm2_ref, rv2_ref,
                    w3_ref, b3_ref, out_ref):
    b = pl.program_id(0)
    for s in range(SPB):
        xb = x_ref[s].astype(jnp.bfloat16)
        acc = jnp.zeros((HW, EMBD), jnp.float32)
        for k in range(NSEL):
            pair = NSEL * (SPB * b + s) + k
            e = idx_ref[pair]
            wk = gwt_ref[pair]
            h = jax.lax.dot_general(
                xb, w1_ref[e], (((0,), (1,)), ((), ())),
                preferred_element_type=jnp.float32)
            sc = g1_ref[pl.ds(e, 1), :] * jax.lax.rsqrt(rv1_ref[pl.ds(e, 1), :] + EPS)
            sh = (b1_ref[pl.ds(e, 1), :] - rm1_ref[pl.ds(e, 1), :]) * sc + be1_ref[pl.ds(e, 1), :]
            h = jnp.maximum(h * sc + sh, 0.0).astype(jnp.bfloat16)
            h = jax.lax.dot_general(
                h, w2_ref[e], (((1,), (1,)), ((), ())),
                preferred_element_type=jnp.float32)
            sc2 = g2_ref[pl.ds(e, 1), :] * jax.lax.rsqrt(rv2_ref[pl.ds(e, 1), :] + EPS)
            sh2 = (b2_ref[pl.ds(e, 1), :] - rm2_ref[pl.ds(e, 1), :]) * sc2 + be2_ref[pl.ds(e, 1), :]
            h = jnp.maximum(h * sc2 + sh2, 0.0).astype(jnp.bfloat16)
            h = jax.lax.dot_general(
                h, w3_ref[e], (((1,), (1,)), ((), ())),
                preferred_element_type=jnp.float32)
            h = h + b3_ref[pl.ds(e, 1), :]
            acc = acc + wk * h
        out_ref[s] = acc


def kernel(x, gate_w, gate_b, w1, b1, g1, be1, rm1, rv1,
           w2, b2, g2, be2, rm2, rv2, w3, b3):
    xr = x.reshape(BATCH, EMBD, HW)
    idx, wts, aux = pl.pallas_call(
        _gate_kernel,
        out_shape=[
            jax.ShapeDtypeStruct((BATCH, NSEL), jnp.int32),
            jax.ShapeDtypeStruct((BATCH, NSEL), jnp.float32),
            jax.ShapeDtypeStruct((1, 1), jnp.float32),
        ],
    )(xr, gate_w, gate_b.reshape(1, NEXP))

    full = lambda shp: pl.BlockSpec(shp, lambda b, i_ref, w_ref: (0,) * len(shp))
    outt = pl.pallas_call(
        _experts_kernel,
        grid_spec=pltpu.PrefetchScalarGridSpec(
            num_scalar_prefetch=2,
            grid=(BATCH // SPB,),
            in_specs=[
                pl.BlockSpec((SPB, EMBD, HW), lambda b, i_ref, w_ref: (b, 0, 0)),
                full((NEXP, HIDD, EMBD)),
                full((NEXP, HIDD)), full((NEXP, HIDD)), full((NEXP, HIDD)),
                full((NEXP, HIDD)), full((NEXP, HIDD)),
                full((NEXP, HIDD, HIDD)),
                full((NEXP, HIDD)), full((NEXP, HIDD)), full((NEXP, HIDD)),
                full((NEXP, HIDD)), full((NEXP, HIDD)),
                full((NEXP, EMBD, HIDD)),
                full((NEXP, EMBD)),
            ],
            out_specs=pl.BlockSpec((SPB, HW, EMBD), lambda b, i_ref, w_ref: (b, 0, 0)),
        ),
        out_shape=jax.ShapeDtypeStruct((BATCH, HW, EMBD), jnp.float32),
    )(idx.reshape(-1), wts.reshape(-1), xr,
      w1.astype(jnp.bfloat16), b1, g1, be1, rm1, rv1,
      w2.astype(jnp.bfloat16), b2, g2, be2, rm2, rv2,
      w3.astype(jnp.bfloat16), b3)

    out = outt.transpose(0, 2, 1).reshape(BATCH, EMBD, HGT, WID)
    return out, aux[0, 0]


# speedup vs baseline: 1.1241x; 1.1241x over previous
import jax
import jax.numpy as jnp
from jax.experimental import pallas as pl
from jax.experimental.pallas import tpu as pltpu

EMBD = 192
NEXP = 8
NSEL = 2
HIDD = 2 * EMBD
BATCH = 32
HGT = 14
WID = 14
HW = HGT * WID
EPS = 1e-5
SPB = 8


def _gate_kernel(x_ref, gw_ref, gb_ref, idx_ref, wts_ref, aux_ref):
    xb = x_ref[...]
    gap = jnp.mean(xb, axis=2)
    logits = jax.lax.dot_general(
        gap, gw_ref[...], (((1,), (1,)), ((), ())),
        preferred_element_type=jnp.float32)
    logits = logits + gb_ref[...]
    m = jnp.max(logits, axis=1, keepdims=True)
    p = jnp.exp(logits - m)
    p = p / jnp.sum(p, axis=1, keepdims=True)
    iota = jax.lax.broadcasted_iota(jnp.int32, (BATCH, NEXP), 1)
    v1 = jnp.max(p, axis=1, keepdims=True)
    i1 = jnp.min(jnp.where(p == v1, iota, NEXP), axis=1, keepdims=True)
    p2 = jnp.where(iota == i1, -1.0, p)
    v2 = jnp.max(p2, axis=1, keepdims=True)
    i2 = jnp.min(jnp.where(p2 == v2, iota, NEXP), axis=1, keepdims=True)
    denom = v1 + v2 + 1e-8
    wa = v1 / denom
    wb = v2 / denom
    idx_ref[:, 0:1] = i1
    idx_ref[:, 1:2] = i2
    wts_ref[:, 0:1] = wa
    wts_ref[:, 1:2] = wb
    gated = jnp.where(iota == i1, wa, 0.0) + jnp.where(iota == i2, wb, 0.0)
    usage = jnp.sum(gated, axis=0, keepdims=True)
    mu = jnp.mean(usage, axis=1, keepdims=True)
    var = jnp.mean((usage - mu) ** 2, axis=1, keepdims=True)
    aux_ref[...] = var / (mu * mu + 1e-10)


def _experts_kernel(idx_ref, gwt_ref, x_ref,
                    w1_ref, b1_ref, g1_ref, be1_ref, rm1_ref, rv1_ref,
                    w2_ref, b2_ref, g2_ref, be2_ref, rm2_ref, rv2_ref,
                    w3_ref, b3_ref, out_ref):
    b = pl.program_id(0)
    for s in range(SPB):
        xb = x_ref[s]
        acc = jnp.zeros((HW, EMBD), jnp.float32)
        for k in range(NSEL):
            e = idx_ref[SPB * b + s, k]
            wk = gwt_ref[SPB * b + s, k]
            h = jax.lax.dot_general(
                xb, w1_ref[e], (((0,), (1,)), ((), ())),
                preferred_element_type=jnp.float32)
            sc = g1_ref[pl.ds(e, 1), :] * jax.lax.rsqrt(rv1_ref[pl.ds(e, 1), :] + EPS)
            sh = (b1_ref[pl.ds(e, 1), :] - rm1_ref[pl.ds(e, 1), :]) * sc + be1_ref[pl.ds(e, 1), :]
            h = jnp.maximum(h * sc + sh, 0.0)
            h = jax.lax.dot_general(
                h, w2_ref[e], (((1,), (1,)), ((), ())),
                preferred_element_type=jnp.float32)
            sc2 = g2_ref[pl.ds(e, 1), :] * jax.lax.rsqrt(rv2_ref[pl.ds(e, 1), :] + EPS)
            sh2 = (b2_ref[pl.ds(e, 1), :] - rm2_ref[pl.ds(e, 1), :]) * sc2 + be2_ref[pl.ds(e, 1), :]
            h = jnp.maximum(h * sc2 + sh2, 0.0)
            h = jax.lax.dot_general(
                h, w3_ref[e], (((1,), (1,)), ((), ())),
                preferred_element_type=jnp.float32)
            h = h + b3_ref[pl.ds(e, 1), :]
            acc = acc + wk * h
        out_ref[s] = acc


def kernel(x, gate_w, gate_b, w1, b1, g1, be1, rm1, rv1,
           w2, b2, g2, be2, rm2, rv2, w3, b3):
    xr = x.reshape(BATCH, EMBD, HW)
    idx, wts, aux = pl.pallas_call(
        _gate_kernel,
        out_shape=[
            jax.ShapeDtypeStruct((BATCH, NSEL), jnp.int32),
            jax.ShapeDtypeStruct((BATCH, NSEL), jnp.float32),
            jax.ShapeDtypeStruct((1, 1), jnp.float32),
        ],
    )(xr, gate_w, gate_b.reshape(1, NEXP))

    full = lambda shp: pl.BlockSpec(shp, lambda b, i_ref, w_ref: (0,) * len(shp))
    outt = pl.pallas_call(
        _experts_kernel,
        grid_spec=pltpu.PrefetchScalarGridSpec(
            num_scalar_prefetch=2,
            grid=(BATCH // SPB,),
            in_specs=[
                pl.BlockSpec((SPB, EMBD, HW), lambda b, i_ref, w_ref: (b, 0, 0)),
                full((NEXP, HIDD, EMBD)),
                full((NEXP, HIDD)), full((NEXP, HIDD)), full((NEXP, HIDD)),
                full((NEXP, HIDD)), full((NEXP, HIDD)),
                full((NEXP, HIDD, HIDD)),
                full((NEXP, HIDD)), full((NEXP, HIDD)), full((NEXP, HIDD)),
                full((NEXP, HIDD)), full((NEXP, HIDD)),
                full((NEXP, EMBD, HIDD)),
                full((NEXP, EMBD)),
            ],
            out_specs=pl.BlockSpec((SPB, HW, EMBD), lambda b, i_ref, w_ref: (b, 0, 0)),
        ),
        out_shape=jax.ShapeDtypeStruct((BATCH, HW, EMBD), jnp.float32),
        compiler_params=pltpu.CompilerParams(
            dimension_semantics=("parallel",)),
    )(idx, wts, xr,
      w1, b1, g1, be1, rm1, rv1,
      w2, b2, g2, be2, rm2, rv2,
      w3, b3)

    out = outt.transpose(0, 2, 1).reshape(BATCH, EMBD, HGT, WID)
    return out, aux[0, 0]


# speedup vs baseline: 1.1682x; 1.0392x over previous
import jax
import jax.numpy as jnp
from jax.experimental import pallas as pl
from jax.experimental.pallas import tpu as pltpu

EMBD = 192
NEXP = 8
NSEL = 2
HIDD = 2 * EMBD
BATCH = 32
HGT = 14
WID = 14
HW = HGT * WID
EPS = 1e-5
SPB = 8
NSTEP = BATCH // SPB


def _moe_kernel(x_ref, gw_ref, gb_ref,
                w1_ref, b1_ref, g1_ref, be1_ref, rm1_ref, rv1_ref,
                w2_ref, b2_ref, g2_ref, be2_ref, rm2_ref, rv2_ref,
                w3_ref, b3_ref, out_ref, aux_ref, usage_ref):
    b = pl.program_id(0)

    xs = x_ref[...]
    gap = jnp.mean(xs, axis=2)
    logits = jax.lax.dot_general(
        gap, gw_ref[...], (((1,), (1,)), ((), ())),
        preferred_element_type=jnp.float32)
    logits = logits + gb_ref[...]
    m = jnp.max(logits, axis=1, keepdims=True)
    p = jnp.exp(logits - m)
    p = p / jnp.sum(p, axis=1, keepdims=True)
    iota = jax.lax.broadcasted_iota(jnp.int32, (SPB, NEXP), 1)
    v1 = jnp.max(p, axis=1, keepdims=True)
    i1 = jnp.min(jnp.where(p == v1, iota, NEXP), axis=1, keepdims=True)
    p2 = jnp.where(iota == i1, -1.0, p)
    v2 = jnp.max(p2, axis=1, keepdims=True)
    i2 = jnp.min(jnp.where(p2 == v2, iota, NEXP), axis=1, keepdims=True)
    denom = v1 + v2 + 1e-8
    wa = v1 / denom
    wb = v2 / denom

    gated = jnp.where(iota == i1, wa, 0.0) + jnp.where(iota == i2, wb, 0.0)
    usage_step = jnp.sum(gated, axis=0, keepdims=True)

    @pl.when(b == 0)
    def _():
        usage_ref[...] = usage_step

    @pl.when(b > 0)
    def _():
        usage_ref[...] = usage_ref[...] + usage_step

    @pl.when(b == NSTEP - 1)
    def _():
        usage = usage_ref[...]
        mu = jnp.mean(usage, axis=1, keepdims=True)
        var = jnp.mean((usage - mu) ** 2, axis=1, keepdims=True)
        aux_ref[...] = var / (mu * mu + 1e-10)

    for s in range(SPB):
        xb = xs[s]
        acc = jnp.zeros((HW, EMBD), jnp.float32)
        for k in range(NSEL):
            e = (i1 if k == 0 else i2)[s, 0]
            wk = (wa if k == 0 else wb)[s:s + 1, :]
            h = jax.lax.dot_general(
                xb, w1_ref[e], (((0,), (1,)), ((), ())),
                preferred_element_type=jnp.float32)
            sc = g1_ref[pl.ds(e, 1), :] * jax.lax.rsqrt(rv1_ref[pl.ds(e, 1), :] + EPS)
            sh = (b1_ref[pl.ds(e, 1), :] - rm1_ref[pl.ds(e, 1), :]) * sc + be1_ref[pl.ds(e, 1), :]
            h = jnp.maximum(h * sc + sh, 0.0)
            h = jax.lax.dot_general(
                h, w2_ref[e], (((1,), (1,)), ((), ())),
                preferred_element_type=jnp.float32)
            sc2 = g2_ref[pl.ds(e, 1), :] * jax.lax.rsqrt(rv2_ref[pl.ds(e, 1), :] + EPS)
            sh2 = (b2_ref[pl.ds(e, 1), :] - rm2_ref[pl.ds(e, 1), :]) * sc2 + be2_ref[pl.ds(e, 1), :]
            h = jnp.maximum(h * sc2 + sh2, 0.0)
            h = jax.lax.dot_general(
                h, w3_ref[e], (((1,), (1,)), ((), ())),
                preferred_element_type=jnp.float32)
            h = h + b3_ref[pl.ds(e, 1), :]
            acc = acc + wk * h
        out_ref[s] = acc


def kernel(x, gate_w, gate_b, w1, b1, g1, be1, rm1, rv1,
           w2, b2, g2, be2, rm2, rv2, w3, b3):
    xr = x.reshape(BATCH, EMBD, HW)
    full = lambda shp: pl.BlockSpec(shp, lambda b: (0,) * len(shp))
    outt, aux = pl.pallas_call(
        _moe_kernel,
        grid=(NSTEP,),
        in_specs=[
            pl.BlockSpec((SPB, EMBD, HW), lambda b: (b, 0, 0)),
            full((NEXP, EMBD)),
            full((1, NEXP)),
            full((NEXP, HIDD, EMBD)),
            full((NEXP, HIDD)), full((NEXP, HIDD)), full((NEXP, HIDD)),
            full((NEXP, HIDD)), full((NEXP, HIDD)),
            full((NEXP, HIDD, HIDD)),
            full((NEXP, HIDD)), full((NEXP, HIDD)), full((NEXP, HIDD)),
            full((NEXP, HIDD)), full((NEXP, HIDD)),
            full((NEXP, EMBD, HIDD)),
            full((NEXP, EMBD)),
        ],
        out_specs=[
            pl.BlockSpec((SPB, HW, EMBD), lambda b: (b, 0, 0)),
            pl.BlockSpec((1, 1), lambda b: (0, 0)),
        ],
        out_shape=[
            jax.ShapeDtypeStruct((BATCH, HW, EMBD), jnp.float32),
            jax.ShapeDtypeStruct((1, 1), jnp.float32),
        ],
        scratch_shapes=[pltpu.VMEM((1, NEXP), jnp.float32)],
    )(xr, gate_w, gate_b.reshape(1, NEXP),
      w1, b1, g1, be1, rm1, rv1,
      w2, b2, g2, be2, rm2, rv2,
      w3, b3)

    out = outt.transpose(0, 2, 1).reshape(BATCH, EMBD, HGT, WID)
    return out, aux[0, 0]


# speedup vs baseline: 1.1858x; 1.0151x over previous
import jax
import jax.numpy as jnp
from jax.experimental import pallas as pl
from jax.experimental.pallas import tpu as pltpu

EMBD = 192
NEXP = 8
NSEL = 2
HIDD = 2 * EMBD
BATCH = 32
HGT = 14
WID = 14
HW = HGT * WID
EPS = 1e-5
SPB = 16
NSTEP = BATCH // SPB


def _moe_kernel(x_ref, gw_ref, gb_ref,
                w1_ref, b1_ref, g1_ref, be1_ref, rm1_ref, rv1_ref,
                w2_ref, b2_ref, g2_ref, be2_ref, rm2_ref, rv2_ref,
                w3_ref, b3_ref, out_ref, aux_ref, usage_ref):
    b = pl.program_id(0)

    xs = x_ref[...]
    gap = jnp.mean(xs, axis=2)
    logits = jax.lax.dot_general(
        gap, gw_ref[...], (((1,), (1,)), ((), ())),
        preferred_element_type=jnp.float32)
    logits = logits + gb_ref[...]
    m = jnp.max(logits, axis=1, keepdims=True)
    p = jnp.exp(logits - m)
    p = p / jnp.sum(p, axis=1, keepdims=True)
    iota = jax.lax.broadcasted_iota(jnp.int32, (SPB, NEXP), 1)
    v1 = jnp.max(p, axis=1, keepdims=True)
    i1 = jnp.min(jnp.where(p == v1, iota, NEXP), axis=1, keepdims=True)
    p2 = jnp.where(iota == i1, -1.0, p)
    v2 = jnp.max(p2, axis=1, keepdims=True)
    i2 = jnp.min(jnp.where(p2 == v2, iota, NEXP), axis=1, keepdims=True)
    denom = v1 + v2 + 1e-8
    wa = v1 / denom
    wb = v2 / denom

    gated = jnp.where(iota == i1, wa, 0.0) + jnp.where(iota == i2, wb, 0.0)
    usage_step = jnp.sum(gated, axis=0, keepdims=True)

    @pl.when(b == 0)
    def _():
        usage_ref[...] = usage_step

    @pl.when(b > 0)
    def _():
        usage_ref[...] = usage_ref[...] + usage_step

    @pl.when(b == NSTEP - 1)
    def _():
        usage = usage_ref[...]
        mu = jnp.mean(usage, axis=1, keepdims=True)
        var = jnp.mean((usage - mu) ** 2, axis=1, keepdims=True)
        aux_ref[...] = var / (mu * mu + 1e-10)

    for s in range(SPB):
        xb = xs[s]
        acc = jnp.zeros((HW, EMBD), jnp.float32)
        for k in range(NSEL):
            e = (i1 if k == 0 else i2)[s, 0]
            wk = (wa if k == 0 else wb)[s:s + 1, :]
            h = jax.lax.dot_general(
                xb, w1_ref[e], (((0,), (1,)), ((), ())),
                preferred_element_type=jnp.float32)
            sc = g1_ref[pl.ds(e, 1), :] * jax.lax.rsqrt(rv1_ref[pl.ds(e, 1), :] + EPS)
            sh = (b1_ref[pl.ds(e, 1), :] - rm1_ref[pl.ds(e, 1), :]) * sc + be1_ref[pl.ds(e, 1), :]
            h = jnp.maximum(h * sc + sh, 0.0)
            h = jax.lax.dot_general(
                h, w2_ref[e], (((1,), (1,)), ((), ())),
                preferred_element_type=jnp.float32)
            sc2 = g2_ref[pl.ds(e, 1), :] * jax.lax.rsqrt(rv2_ref[pl.ds(e, 1), :] + EPS)
            sh2 = (b2_ref[pl.ds(e, 1), :] - rm2_ref[pl.ds(e, 1), :]) * sc2 + be2_ref[pl.ds(e, 1), :]
            h = jnp.maximum(h * sc2 + sh2, 0.0)
            h = jax.lax.dot_general(
                h, w3_ref[e], (((1,), (1,)), ((), ())),
                preferred_element_type=jnp.float32)
            h = h + b3_ref[pl.ds(e, 1), :]
            acc = acc + wk * h
        out_ref[s] = acc


def kernel(x, gate_w, gate_b, w1, b1, g1, be1, rm1, rv1,
           w2, b2, g2, be2, rm2, rv2, w3, b3):
    xr = x.reshape(BATCH, EMBD, HW)
    full = lambda shp: pl.BlockSpec(shp, lambda b: (0,) * len(shp))
    outt, aux = pl.pallas_call(
        _moe_kernel,
        grid=(NSTEP,),
        in_specs=[
            pl.BlockSpec((SPB, EMBD, HW), lambda b: (b, 0, 0)),
            full((NEXP, EMBD)),
            full((1, NEXP)),
            full((NEXP, HIDD, EMBD)),
            full((NEXP, HIDD)), full((NEXP, HIDD)), full((NEXP, HIDD)),
            full((NEXP, HIDD)), full((NEXP, HIDD)),
            full((NEXP, HIDD, HIDD)),
            full((NEXP, HIDD)), full((NEXP, HIDD)), full((NEXP, HIDD)),
            full((NEXP, HIDD)), full((NEXP, HIDD)),
            full((NEXP, EMBD, HIDD)),
            full((NEXP, EMBD)),
        ],
        out_specs=[
            pl.BlockSpec((SPB, HW, EMBD), lambda b: (b, 0, 0)),
            pl.BlockSpec((1, 1), lambda b: (0, 0)),
        ],
        out_shape=[
            jax.ShapeDtypeStruct((BATCH, HW, EMBD), jnp.float32),
            jax.ShapeDtypeStruct((1, 1), jnp.float32),
        ],
        scratch_shapes=[pltpu.VMEM((1, NEXP), jnp.float32)],
    )(xr, gate_w, gate_b.reshape(1, NEXP),
      w1, b1, g1, be1, rm1, rv1,
      w2, b2, g2, be2, rm2, rv2,
      w3, b3)

    out = outt.transpose(0, 2, 1).reshape(BATCH, EMBD, HGT, WID)
    return out, aux[0, 0]


# speedup vs baseline: 1.2017x; 1.0134x over previous
import jax
import jax.numpy as jnp
from jax.experimental import pallas as pl
from jax.experimental.pallas import tpu as pltpu

EMBD = 192
NEXP = 8
NSEL = 2
HIDD = 2 * EMBD
BATCH = 32
HGT = 14
WID = 14
HW = HGT * WID
EPS = 1e-5
SPB = 32
NSTEP = BATCH // SPB


def _moe_kernel(x_ref, gw_ref, gb_ref,
                w1_ref, b1_ref, g1_ref, be1_ref, rm1_ref, rv1_ref,
                w2_ref, b2_ref, g2_ref, be2_ref, rm2_ref, rv2_ref,
                w3_ref, b3_ref, out_ref, aux_ref, usage_ref):
    b = pl.program_id(0)

    xs = x_ref[...]
    gap = jnp.mean(xs, axis=2)
    logits = jax.lax.dot_general(
        gap, gw_ref[...], (((1,), (1,)), ((), ())),
        preferred_element_type=jnp.float32)
    logits = logits + gb_ref[...]
    m = jnp.max(logits, axis=1, keepdims=True)
    p = jnp.exp(logits - m)
    p = p / jnp.sum(p, axis=1, keepdims=True)
    iota = jax.lax.broadcasted_iota(jnp.int32, (SPB, NEXP), 1)
    v1 = jnp.max(p, axis=1, keepdims=True)
    i1 = jnp.min(jnp.where(p == v1, iota, NEXP), axis=1, keepdims=True)
    p2 = jnp.where(iota == i1, -1.0, p)
    v2 = jnp.max(p2, axis=1, keepdims=True)
    i2 = jnp.min(jnp.where(p2 == v2, iota, NEXP), axis=1, keepdims=True)
    denom = v1 + v2 + 1e-8
    wa = v1 / denom
    wb = v2 / denom

    gated = jnp.where(iota == i1, wa, 0.0) + jnp.where(iota == i2, wb, 0.0)
    usage_step = jnp.sum(gated, axis=0, keepdims=True)

    @pl.when(b == 0)
    def _():
        usage_ref[...] = usage_step

    @pl.when(b > 0)
    def _():
        usage_ref[...] = usage_ref[...] + usage_step

    @pl.when(b == NSTEP - 1)
    def _():
        usage = usage_ref[...]
        mu = jnp.mean(usage, axis=1, keepdims=True)
        var = jnp.mean((usage - mu) ** 2, axis=1, keepdims=True)
        aux_ref[...] = var / (mu * mu + 1e-10)

    for s in range(SPB):
        xb = xs[s]
        acc = jnp.zeros((HW, EMBD), jnp.float32)
        for k in range(NSEL):
            e = (i1 if k == 0 else i2)[s, 0]
            wk = (wa if k == 0 else wb)[s:s + 1, :]
            h = jax.lax.dot_general(
                xb, w1_ref[e], (((0,), (1,)), ((), ())),
                preferred_element_type=jnp.float32)
            sc = g1_ref[pl.ds(e, 1), :] * jax.lax.rsqrt(rv1_ref[pl.ds(e, 1), :] + EPS)
            sh = (b1_ref[pl.ds(e, 1), :] - rm1_ref[pl.ds(e, 1), :]) * sc + be1_ref[pl.ds(e, 1), :]
            h = jnp.maximum(h * sc + sh, 0.0)
            h = jax.lax.dot_general(
                h, w2_ref[e], (((1,), (1,)), ((), ())),
                preferred_element_type=jnp.float32)
            sc2 = g2_ref[pl.ds(e, 1), :] * jax.lax.rsqrt(rv2_ref[pl.ds(e, 1), :] + EPS)
            sh2 = (b2_ref[pl.ds(e, 1), :] - rm2_ref[pl.ds(e, 1), :]) * sc2 + be2_ref[pl.ds(e, 1), :]
            h = jnp.maximum(h * sc2 + sh2, 0.0)
            h = jax.lax.dot_general(
                h, w3_ref[e], (((1,), (1,)), ((), ())),
                preferred_element_type=jnp.float32)
            h = h + b3_ref[pl.ds(e, 1), :]
            acc = acc + wk * h
        out_ref[s] = acc


def kernel(x, gate_w, gate_b, w1, b1, g1, be1, rm1, rv1,
           w2, b2, g2, be2, rm2, rv2, w3, b3):
    xr = x.reshape(BATCH, EMBD, HW)
    full = lambda shp: pl.BlockSpec(shp, lambda b: (0,) * len(shp))
    outt, aux = pl.pallas_call(
        _moe_kernel,
        grid=(NSTEP,),
        in_specs=[
            pl.BlockSpec((SPB, EMBD, HW), lambda b: (b, 0, 0)),
            full((NEXP, EMBD)),
            full((1, NEXP)),
            full((NEXP, HIDD, EMBD)),
            full((NEXP, HIDD)), full((NEXP, HIDD)), full((NEXP, HIDD)),
            full((NEXP, HIDD)), full((NEXP, HIDD)),
            full((NEXP, HIDD, HIDD)),
            full((NEXP, HIDD)), full((NEXP, HIDD)), full((NEXP, HIDD)),
            full((NEXP, HIDD)), full((NEXP, HIDD)),
            full((NEXP, EMBD, HIDD)),
            full((NEXP, EMBD)),
        ],
        out_specs=[
            pl.BlockSpec((SPB, HW, EMBD), lambda b: (b, 0, 0)),
            pl.BlockSpec((1, 1), lambda b: (0, 0)),
        ],
        out_shape=[
            jax.ShapeDtypeStruct((BATCH, HW, EMBD), jnp.float32),
            jax.ShapeDtypeStruct((1, 1), jnp.float32),
        ],
        scratch_shapes=[pltpu.VMEM((1, NEXP), jnp.float32)],
    )(xr, gate_w, gate_b.reshape(1, NEXP),
      w1, b1, g1, be1, rm1, rv1,
      w2, b2, g2, be2, rm2, rv2,
      w3, b3)

    out = outt.transpose(0, 2, 1).reshape(BATCH, EMBD, HGT, WID)
    return out, aux[0, 0]


# speedup vs baseline: 1.4356x; 1.1946x over previous
import jax
import jax.numpy as jnp
from jax.experimental import pallas as pl
from jax.experimental.pallas import tpu as pltpu

EMBD = 192
NEXP = 8
NSEL = 2
HIDD = 2 * EMBD
BATCH = 32
HGT = 14
WID = 14
HW = HGT * WID
EPS = 1e-5
SPB = 32
NSTEP = BATCH // SPB


def _moe_kernel(x_ref, gw_ref, gb_ref,
                w1_ref, b1_ref, g1_ref, be1_ref, rm1_ref, rv1_ref,
                w2_ref, b2_ref, g2_ref, be2_ref, rm2_ref, rv2_ref,
                w3_ref, b3_ref, out_ref, aux_ref, usage_ref):
    b = pl.program_id(0)

    gap = jnp.mean(x_ref[...], axis=0)
    logits = jax.lax.dot_general(
        gap, gw_ref[...], (((1,), (1,)), ((), ())),
        preferred_element_type=jnp.float32)
    logits = logits + gb_ref[...]
    m = jnp.max(logits, axis=1, keepdims=True)
    p = jnp.exp(logits - m)
    p = p / jnp.sum(p, axis=1, keepdims=True)
    iota = jax.lax.broadcasted_iota(jnp.int32, (SPB, NEXP), 1)
    v1 = jnp.max(p, axis=1, keepdims=True)
    i1 = jnp.min(jnp.where(p == v1, iota, NEXP), axis=1, keepdims=True)
    p2 = jnp.where(iota == i1, -1.0, p)
    v2 = jnp.max(p2, axis=1, keepdims=True)
    i2 = jnp.min(jnp.where(p2 == v2, iota, NEXP), axis=1, keepdims=True)
    denom = v1 + v2 + 1e-8
    wa = v1 / denom
    wb = v2 / denom

    gated = jnp.where(iota == i1, wa, 0.0) + jnp.where(iota == i2, wb, 0.0)
    usage_step = jnp.sum(gated, axis=0, keepdims=True)

    @pl.when(b == 0)
    def _():
        usage_ref[...] = usage_step

    @pl.when(b > 0)
    def _():
        usage_ref[...] = usage_ref[...] + usage_step

    @pl.when(b == NSTEP - 1)
    def _():
        usage = usage_ref[...]
        mu = jnp.mean(usage, axis=1, keepdims=True)
        var = jnp.mean((usage - mu) ** 2, axis=1, keepdims=True)
        aux_ref[...] = var / (mu * mu + 1e-10)

    for s in range(SPB):
        xb = x_ref[:, s, :]
        acc = jnp.zeros((HW, EMBD), jnp.float32)
        for k in range(NSEL):
            e = (i1 if k == 0 else i2)[s, 0]
            wk = (wa if k == 0 else wb)[s:s + 1, :]
            h = jax.lax.dot_general(
                xb, w1_ref[e], (((1,), (1,)), ((), ())),
                preferred_element_type=jnp.float32)
            sc = g1_ref[pl.ds(e, 1), :] * jax.lax.rsqrt(rv1_ref[pl.ds(e, 1), :] + EPS)
            sh = (b1_ref[pl.ds(e, 1), :] - rm1_ref[pl.ds(e, 1), :]) * sc + be1_ref[pl.ds(e, 1), :]
            h = jnp.maximum(h * sc + sh, 0.0)
            h = jax.lax.dot_general(
                h, w2_ref[e], (((1,), (1,)), ((), ())),
                preferred_element_type=jnp.float32)
            sc2 = g2_ref[pl.ds(e, 1), :] * jax.lax.rsqrt(rv2_ref[pl.ds(e, 1), :] + EPS)
            sh2 = (b2_ref[pl.ds(e, 1), :] - rm2_ref[pl.ds(e, 1), :]) * sc2 + be2_ref[pl.ds(e, 1), :]
            h = jnp.maximum(h * sc2 + sh2, 0.0)
            h = jax.lax.dot_general(
                h, w3_ref[e], (((1,), (1,)), ((), ())),
                preferred_element_type=jnp.float32)
            h = h + b3_ref[pl.ds(e, 1), :]
            acc = acc + wk * h
        out_ref[s] = acc


def kernel(x, gate_w, gate_b, w1, b1, g1, be1, rm1, rv1,
           w2, b2, g2, be2, rm2, rv2, w3, b3):
    xr = x.transpose(2, 3, 0, 1).reshape(HW, BATCH, EMBD)
    full = lambda shp: pl.BlockSpec(shp, lambda b: (0,) * len(shp))
    outt, aux = pl.pallas_call(
        _moe_kernel,
        grid=(NSTEP,),
        in_specs=[
            pl.BlockSpec((HW, SPB, EMBD), lambda b: (0, b, 0)),
            full((NEXP, EMBD)),
            full((1, NEXP)),
            full((NEXP, HIDD, EMBD)),
            full((NEXP, HIDD)), full((NEXP, HIDD)), full((NEXP, HIDD)),
            full((NEXP, HIDD)), full((NEXP, HIDD)),
            full((NEXP, HIDD, HIDD)),
            full((NEXP, HIDD)), full((NEXP, HIDD)), full((NEXP, HIDD)),
            full((NEXP, HIDD)), full((NEXP, HIDD)),
            full((NEXP, EMBD, HIDD)),
            full((NEXP, EMBD)),
        ],
        out_specs=[
            pl.BlockSpec((SPB, HW, EMBD), lambda b: (b, 0, 0)),
            pl.BlockSpec((1, 1), lambda b: (0, 0)),
        ],
        out_shape=[
            jax.ShapeDtypeStruct((BATCH, HW, EMBD), jnp.float32),
            jax.ShapeDtypeStruct((1, 1), jnp.float32),
        ],
        scratch_shapes=[pltpu.VMEM((1, NEXP), jnp.float32)],
    )(xr, gate_w, gate_b.reshape(1, NEXP),
      w1, b1, g1, be1, rm1, rv1,
      w2, b2, g2, be2, rm2, rv2,
      w3, b3)

    out = outt.transpose(0, 2, 1).reshape(BATCH, EMBD, HGT, WID)
    return out, aux[0, 0]


# speedup vs baseline: 1.5488x; 1.0788x over previous
import jax
import jax.numpy as jnp
from jax.experimental import pallas as pl
from jax.experimental.pallas import tpu as pltpu

EMBD = 192
NEXP = 8
NSEL = 2
HIDD = 2 * EMBD
BATCH = 32
HGT = 14
WID = 14
HW = HGT * WID
EPS = 1e-5
SPB = 32
NSTEP = BATCH // SPB


def _moe_kernel(x_ref, gw_ref, gb_ref,
                w1_ref, b1_ref, g1_ref, be1_ref, rm1_ref, rv1_ref,
                w2_ref, b2_ref, g2_ref, be2_ref, rm2_ref, rv2_ref,
                w3_ref, b3_ref, out_ref, aux_ref, usage_ref):
    b = pl.program_id(0)

    gap = jnp.mean(x_ref[...], axis=0)
    logits = jax.lax.dot_general(
        gap, gw_ref[...], (((1,), (1,)), ((), ())),
        preferred_element_type=jnp.float32)
    logits = logits + gb_ref[...]
    m = jnp.max(logits, axis=1, keepdims=True)
    p = jnp.exp(logits - m)
    p = p / jnp.sum(p, axis=1, keepdims=True)
    iota = jax.lax.broadcasted_iota(jnp.int32, (SPB, NEXP), 1)
    v1 = jnp.max(p, axis=1, keepdims=True)
    i1 = jnp.min(jnp.where(p == v1, iota, NEXP), axis=1, keepdims=True)
    p2 = jnp.where(iota == i1, -1.0, p)
    v2 = jnp.max(p2, axis=1, keepdims=True)
    i2 = jnp.min(jnp.where(p2 == v2, iota, NEXP), axis=1, keepdims=True)
    denom = v1 + v2 + 1e-8
    wa = v1 / denom
    wb = v2 / denom

    gated = jnp.where(iota == i1, wa, 0.0) + jnp.where(iota == i2, wb, 0.0)
    usage_step = jnp.sum(gated, axis=0, keepdims=True)

    @pl.when(b == 0)
    def _():
        usage_ref[...] = usage_step

    @pl.when(b > 0)
    def _():
        usage_ref[...] = usage_ref[...] + usage_step

    @pl.when(b == NSTEP - 1)
    def _():
        usage = usage_ref[...]
        mu = jnp.mean(usage, axis=1, keepdims=True)
        var = jnp.mean((usage - mu) ** 2, axis=1, keepdims=True)
        aux_ref[...] = var / (mu * mu + 1e-10)

    for s in range(SPB):
        xb = x_ref[:, s, :]
        acc = jnp.zeros((HW, EMBD), jnp.float32)
        for k in range(NSEL):
            e = (i1 if k == 0 else i2)[s, 0]
            wk = (wa if k == 0 else wb)[s:s + 1, :]
            h = jax.lax.dot_general(
                xb, w1_ref[e], (((1,), (0,)), ((), ())),
                preferred_element_type=jnp.float32)
            sc = g1_ref[pl.ds(e, 1), :] * jax.lax.rsqrt(rv1_ref[pl.ds(e, 1), :] + EPS)
            sh = (b1_ref[pl.ds(e, 1), :] - rm1_ref[pl.ds(e, 1), :]) * sc + be1_ref[pl.ds(e, 1), :]
            h = jnp.maximum(h * sc + sh, 0.0)
            h = jax.lax.dot_general(
                h, w2_ref[e], (((1,), (1,)), ((), ())),
                preferred_element_type=jnp.float32)
            sc2 = g2_ref[pl.ds(e, 1), :] * jax.lax.rsqrt(rv2_ref[pl.ds(e, 1), :] + EPS)
            sh2 = (b2_ref[pl.ds(e, 1), :] - rm2_ref[pl.ds(e, 1), :]) * sc2 + be2_ref[pl.ds(e, 1), :]
            h = jnp.maximum(h * sc2 + sh2, 0.0)
            h = jax.lax.dot_general(
                h, w3_ref[e], (((1,), (1,)), ((), ())),
                preferred_element_type=jnp.float32)
            h = h + b3_ref[pl.ds(e, 1), :]
            acc = acc + wk * h
        out_ref[s] = acc


def kernel(x, gate_w, gate_b, w1, b1, g1, be1, rm1, rv1,
           w2, b2, g2, be2, rm2, rv2, w3, b3):
    xr = x.transpose(2, 3, 0, 1).reshape(HW, BATCH, EMBD)
    full = lambda shp: pl.BlockSpec(shp, lambda b: (0,) * len(shp))
    outt, aux = pl.pallas_call(
        _moe_kernel,
        grid=(NSTEP,),
        in_specs=[
            pl.BlockSpec((HW, SPB, EMBD), lambda b: (0, b, 0)),
            full((NEXP, EMBD)),
            full((1, NEXP)),
            full((NEXP, EMBD, HIDD)),
            full((NEXP, HIDD)), full((NEXP, HIDD)), full((NEXP, HIDD)),
            full((NEXP, HIDD)), full((NEXP, HIDD)),
            full((NEXP, HIDD, HIDD)),
            full((NEXP, HIDD)), full((NEXP, HIDD)), full((NEXP, HIDD)),
            full((NEXP, HIDD)), full((NEXP, HIDD)),
            full((NEXP, EMBD, HIDD)),
            full((NEXP, EMBD)),
        ],
        out_specs=[
            pl.BlockSpec((SPB, HW, EMBD), lambda b: (b, 0, 0)),
            pl.BlockSpec((1, 1), lambda b: (0, 0)),
        ],
        out_shape=[
            jax.ShapeDtypeStruct((BATCH, HW, EMBD), jnp.float32),
            jax.ShapeDtypeStruct((1, 1), jnp.float32),
        ],
        scratch_shapes=[pltpu.VMEM((1, NEXP), jnp.float32)],
    )(xr, gate_w, gate_b.reshape(1, NEXP),
      w1.transpose(0, 2, 1), b1, g1, be1, rm1, rv1,
      w2, b2, g2, be2, rm2, rv2,
      w3, b3)

    out = outt.transpose(0, 2, 1).reshape(BATCH, EMBD, HGT, WID)
    return out, aux[0, 0]


# speedup vs baseline: 1.7780x; 1.1480x over previous
import jax
import jax.numpy as jnp
from jax.experimental import pallas as pl
from jax.experimental.pallas import tpu as pltpu

EMBD = 192
NEXP = 8
NSEL = 2
HIDD = 2 * EMBD
BATCH = 32
HGT = 14
WID = 14
HW = HGT * WID
EPS = 1e-5
SPB = 32
NSTEP = BATCH // SPB


def _moe_kernel(x_ref, gw_ref, gb_ref,
                w1_ref, b1_ref, g1_ref, be1_ref, rm1_ref, rv1_ref,
                w2_ref, b2_ref, g2_ref, be2_ref, rm2_ref, rv2_ref,
                w3_ref, b3_ref, out_ref, aux_ref, usage_ref):
    b = pl.program_id(0)

    gap = jnp.mean(x_ref[...], axis=0)
    logits = jax.lax.dot_general(
        gap, gw_ref[...], (((1,), (1,)), ((), ())),
        preferred_element_type=jnp.float32)
    logits = logits + gb_ref[...]
    m = jnp.max(logits, axis=1, keepdims=True)
    p = jnp.exp(logits - m)
    p = p / jnp.sum(p, axis=1, keepdims=True)
    iota = jax.lax.broadcasted_iota(jnp.int32, (SPB, NEXP), 1)
    v1 = jnp.max(p, axis=1, keepdims=True)
    i1 = jnp.min(jnp.where(p == v1, iota, NEXP), axis=1, keepdims=True)
    p2 = jnp.where(iota == i1, -1.0, p)
    v2 = jnp.max(p2, axis=1, keepdims=True)
    i2 = jnp.min(jnp.where(p2 == v2, iota, NEXP), axis=1, keepdims=True)
    denom = v1 + v2 + 1e-8
    wa = v1 / denom
    wb = v2 / denom

    gated = jnp.where(iota == i1, wa, 0.0) + jnp.where(iota == i2, wb, 0.0)
    usage_step = jnp.sum(gated, axis=0, keepdims=True)

    @pl.when(b == 0)
    def _():
        usage_ref[...] = usage_step

    @pl.when(b > 0)
    def _():
        usage_ref[...] = usage_ref[...] + usage_step

    @pl.when(b == NSTEP - 1)
    def _():
        usage = usage_ref[...]
        mu = jnp.mean(usage, axis=1, keepdims=True)
        var = jnp.mean((usage - mu) ** 2, axis=1, keepdims=True)
        aux_ref[...] = var / (mu * mu + 1e-10)

    for s in range(SPB):
        xb = x_ref[:, s, :]
        acc = jnp.zeros((HW, EMBD), jnp.float32)
        for k in range(NSEL):
            e = (i1 if k == 0 else i2)[s, 0]
            wk = (wa if k == 0 else wb)[s:s + 1, :]
            h = jax.lax.dot_general(
                xb, w1_ref[e], (((1,), (0,)), ((), ())),
                preferred_element_type=jnp.float32)
            sc = g1_ref[pl.ds(e, 1), :] * jax.lax.rsqrt(rv1_ref[pl.ds(e, 1), :] + EPS)
            sh = (b1_ref[pl.ds(e, 1), :] - rm1_ref[pl.ds(e, 1), :]) * sc + be1_ref[pl.ds(e, 1), :]
            h = jnp.maximum(h * sc + sh, 0.0)
            h = jax.lax.dot_general(
                h, w2_ref[e], (((1,), (1,)), ((), ())),
                preferred_element_type=jnp.float32)
            sc2 = g2_ref[pl.ds(e, 1), :] * jax.lax.rsqrt(rv2_ref[pl.ds(e, 1), :] + EPS)
            sh2 = (b2_ref[pl.ds(e, 1), :] - rm2_ref[pl.ds(e, 1), :]) * sc2 + be2_ref[pl.ds(e, 1), :]
            h = jnp.maximum(h * sc2 + sh2, 0.0)
            h = jax.lax.dot_general(
                h, w3_ref[e], (((1,), (1,)), ((), ())),
                preferred_element_type=jnp.float32)
            h = h + b3_ref[pl.ds(e, 1), :]
            acc = acc + wk * h
        out_ref[:, s, :] = acc


def kernel(x, gate_w, gate_b, w1, b1, g1, be1, rm1, rv1,
           w2, b2, g2, be2, rm2, rv2, w3, b3):
    xr = x.transpose(2, 3, 0, 1).reshape(HW, BATCH, EMBD)
    full = lambda shp: pl.BlockSpec(shp, lambda b: (0,) * len(shp))
    outt, aux = pl.pallas_call(
        _moe_kernel,
        grid=(NSTEP,),
        in_specs=[
            pl.BlockSpec((HW, SPB, EMBD), lambda b: (0, b, 0)),
            full((NEXP, EMBD)),
            full((1, NEXP)),
            full((NEXP, EMBD, HIDD)),
            full((NEXP, HIDD)), full((NEXP, HIDD)), full((NEXP, HIDD)),
            full((NEXP, HIDD)), full((NEXP, HIDD)),
            full((NEXP, HIDD, HIDD)),
            full((NEXP, HIDD)), full((NEXP, HIDD)), full((NEXP, HIDD)),
            full((NEXP, HIDD)), full((NEXP, HIDD)),
            full((NEXP, EMBD, HIDD)),
            full((NEXP, EMBD)),
        ],
        out_specs=[
            pl.BlockSpec((HW, SPB, EMBD), lambda b: (0, b, 0)),
            pl.BlockSpec((1, 1), lambda b: (0, 0)),
        ],
        out_shape=[
            jax.ShapeDtypeStruct((HW, BATCH, EMBD), jnp.float32),
            jax.ShapeDtypeStruct((1, 1), jnp.float32),
        ],
        scratch_shapes=[pltpu.VMEM((1, NEXP), jnp.float32)],
    )(xr, gate_w, gate_b.reshape(1, NEXP),
      w1.transpose(0, 2, 1), b1, g1, be1, rm1, rv1,
      w2, b2, g2, be2, rm2, rv2,
      w3, b3)

    out = outt.reshape(HGT, WID, BATCH, EMBD).transpose(2, 3, 0, 1)
    return out, aux[0, 0]
